# baseline (speedup 1.0000x reference)
import functools

import jax
import jax.numpy as jnp
from jax import lax
from jax.experimental import pallas as pl
from jax.experimental.pallas import tpu as pltpu
from jax.experimental.pallas import tpu_sc as plsc
from jax.experimental import layout as jlayout

B = 16384
V = 32768
NB = 15
NC = 2
NS = 16
NW = NC * NS
BPW = B // NW
CH = 32
NCH = BPW // CH


def _pack_body(x_ref, lin_ref):
    i = lax.broadcasted_iota(jnp.int32, (NB, 1), 0)
    w = jnp.left_shift(jnp.int32(1), NB - 1 - i)
    lin_ref[...] = jnp.sum(x_ref[...] * w, axis=0)


_pack = pl.pallas_call(
    _pack_body,
    out_shape=jax.ShapeDtypeStruct((B,), jnp.int32),
)


NBUF = 3


def _gbody(params_hbm, lin_hbm, out_hbm, idx_v, buf0, buf1, buf2,
           gsem0, gsem1, gsem2, osem0, osem1, osem2):
    wid = lax.axis_index("s") * NC + lax.axis_index("c")
    base = wid * BPW

    pltpu.sync_copy(lin_hbm.at[pl.ds(base, BPW)], idx_v)

    bufs = (buf0, buf1, buf2)
    gsems = (gsem0, gsem1, gsem2)
    osems = (osem0, osem1, osem2)

    def fire(j, b):
        for g in range(CH // 16):
            v = idx_v[pl.ds(j * CH + g * 16, 16)]
            for l in range(16):
                pltpu.async_copy(params_hbm.at[v[l]],
                                 bufs[b].at[g * 16 + l], gsems[b])

    def drain_gather(b):
        pltpu.make_async_copy(params_hbm.at[pl.ds(0, CH)], bufs[b],
                              gsems[b]).wait()

    def start_out(j, b):
        pltpu.async_copy(bufs[b], out_hbm.at[pl.ds(base + j * CH, CH)],
                         osems[b])

    def wait_out(j, b):
        pltpu.make_async_copy(bufs[b], out_hbm.at[pl.ds(base + j * CH, CH)],
                              osems[b]).wait()

    fire(0, 0)
    for j in range(NCH):
        if j + 1 < NCH:
            if j - 2 >= 0:
                wait_out(j - 2, (j - 2) % NBUF)
            fire(j + 1, (j + 1) % NBUF)
        drain_gather(j % NBUF)
        start_out(j, j % NBUF)
    wait_out(NCH - 2, (NCH - 2) % NBUF)
    wait_out(NCH - 1, (NCH - 1) % NBUF)


_gather = functools.partial(
    pl.kernel,
    mesh=plsc.VectorSubcoreMesh(core_axis_name="c", subcore_axis_name="s"),
    out_type=jax.ShapeDtypeStruct((B, 8, 64), jnp.float32),
    scratch_types=[
        pltpu.VMEM((BPW,), jnp.int32),
        pltpu.VMEM((CH, 8, 64), jnp.float32),
        pltpu.VMEM((CH, 8, 64), jnp.float32),
        pltpu.VMEM((CH, 8, 64), jnp.float32),
        pltpu.SemaphoreType.DMA,
        pltpu.SemaphoreType.DMA,
        pltpu.SemaphoreType.DMA,
        pltpu.SemaphoreType.DMA,
        pltpu.SemaphoreType.DMA,
        pltpu.SemaphoreType.DMA,
    ],
    compiler_params=pltpu.CompilerParams(use_tc_tiling_on_sc=True),
)(_gbody)


def _run(x, params):
    lin = _pack(x)
    table = params.reshape(V, 8, 64)
    return _gather(table, lin)


@functools.lru_cache(maxsize=None)
def _jitted(dev):
    fmt = jlayout.Format(
        jlayout.Layout(major_to_minor=(0, 1, 2)),
        jax.sharding.SingleDeviceSharding(dev))
    return jax.jit(_run, out_shardings=fmt)


def kernel(x, params):
    try:
        dev = next(iter(params.devices()))
    except Exception:
        return _run(x, params)
    return _jitted(dev)(x, params)


# speedup vs baseline: 1.0068x; 1.0068x over previous
import functools

import jax
import jax.numpy as jnp
from jax import lax
from jax.experimental import pallas as pl
from jax.experimental.pallas import tpu as pltpu
from jax.experimental.pallas import tpu_sc as plsc
from jax.experimental import layout as jlayout

B = 16384
V = 32768
NB = 15
NC = 2
NS = 16
NW = NC * NS
BPW = B // NW
CH = 32
NCH = BPW // CH


def _pack_body(x_ref, lin_ref):
    i = lax.broadcasted_iota(jnp.int32, (NB, 1), 0)
    w = jnp.left_shift(jnp.int32(1), NB - 1 - i)
    lin_ref[...] = jnp.sum(x_ref[...] * w, axis=0)


_pack = pl.pallas_call(
    _pack_body,
    out_shape=jax.ShapeDtypeStruct((B,), jnp.int32),
)


NBUF = 3


def _gbody(params_hbm, lin_hbm, out_hbm, idx_v, buf0, buf1, buf2,
           gsem0, gsem1, gsem2, osem0, osem1, osem2):
    wid = lax.axis_index("s") * NC + lax.axis_index("c")
    base = wid * BPW

    pltpu.sync_copy(lin_hbm.at[pl.ds(base, BPW)], idx_v)

    bufs = (buf0, buf1, buf2)
    gsems = (gsem0, gsem1, gsem2)
    osems = (osem0, osem1, osem2)

    def fire(j, b):
        for g in range(CH // 16):
            v = idx_v[pl.ds(j * CH + g * 16, 16)]
            for l in range(16):
                pltpu.async_copy(params_hbm.at[v[l]],
                                 bufs[b].at[g * 16 + l], gsems[b])

    def drain_gather(b):
        pltpu.make_async_copy(params_hbm.at[pl.ds(0, CH)], bufs[b],
                              gsems[b]).wait()

    def start_out(j, b):
        pltpu.async_copy(bufs[b], out_hbm.at[pl.ds(base + j * CH, CH)],
                         osems[b])

    def wait_out(j, b):
        pltpu.make_async_copy(bufs[b], out_hbm.at[pl.ds(base + j * CH, CH)],
                              osems[b]).wait()

    fire(0, 0)
    for j in range(NCH):
        if j + 1 < NCH:
            if j - 2 >= 0:
                wait_out(j - 2, (j - 2) % NBUF)
            fire(j + 1, (j + 1) % NBUF)
        drain_gather(j % NBUF)
        start_out(j, j % NBUF)
    wait_out(NCH - 2, (NCH - 2) % NBUF)
    wait_out(NCH - 1, (NCH - 1) % NBUF)


_gather = functools.partial(
    pl.kernel,
    mesh=plsc.VectorSubcoreMesh(core_axis_name="c", subcore_axis_name="s"),
    out_type=jax.ShapeDtypeStruct((B, 8, 64), jnp.float32),
    scratch_types=[
        pltpu.VMEM((BPW,), jnp.int32),
        pltpu.VMEM((CH, 8, 64), jnp.float32),
        pltpu.VMEM((CH, 8, 64), jnp.float32),
        pltpu.VMEM((CH, 8, 64), jnp.float32),
        pltpu.SemaphoreType.DMA,
        pltpu.SemaphoreType.DMA,
        pltpu.SemaphoreType.DMA,
        pltpu.SemaphoreType.DMA,
        pltpu.SemaphoreType.DMA,
        pltpu.SemaphoreType.DMA,
    ],
    compiler_params=pltpu.CompilerParams(use_tc_tiling_on_sc=True),
)(_gbody)


def kernel(x, params):
    lin = _pack(x)
    table = params.reshape(V, 8, 64)
    out = _gather(table, lin)
    try:
        fmt = jlayout.Format(
            jlayout.Layout(major_to_minor=(0, 1, 2)),
            jax.sharding.SingleDeviceSharding(jax.devices()[0]))
        out = jlayout.with_layout_constraint(out, fmt)
    except Exception:
        pass
    return out
